# Initial kernel scaffold; baseline (speedup 1.0000x reference)
import jax
import jax.numpy as jnp
from jax.experimental import pallas as pl


def kernel(pos_edge, neg_edge, entity_emb, relation_emb):
    raise NotImplementedError("write your pallas kernel here")


# speedup vs baseline: 1.7495x; 1.7495x over previous
import jax
import jax.numpy as jnp
from jax import lax
from jax.experimental import pallas as pl
from jax.experimental.pallas import tpu as pltpu
from jax.experimental.pallas import tpu_sc as plsc

_EMB_DIM = 128
_BATCH = 4096
_PAD = 100352
_TILES = 32
_EPT = _BATCH // _TILES
_SLICE = _PAD // 16
_GROUPS = _EMB_DIM // 16


def _sc_body(posF, negF, ent, rel,
             posd_o, negd_o, esum_o, ecnt_o, rsum_o, rcnt_o,
             i_ph, i_pr, i_pt, i_nh, i_nr, i_nt,
             rows_h, rows_r, rows_t,
             v_d, v_h, v_t, v_r, ones_v, zbuf,
             esum_s, ecnt_s, rsum_s, rcnt_s):
    c = lax.axis_index("c")
    s = lax.axis_index("s")
    wid = c * 16 + s
    base = wid * _EPT

    zero16 = jnp.zeros((16,), jnp.float32)
    one16 = jnp.ones((16,), jnp.float32)

    def zfill(i, carry):
        zbuf[pl.ds(i * 16, 16)] = zero16
        return carry

    lax.fori_loop(0, _SLICE // 16, zfill, None)
    for i in range(_EPT // 16):
        ones_v[pl.ds(i * 16, 16)] = one16

    off = s * _SLICE
    pltpu.sync_copy(zbuf, esum_s.at[pl.ds(off, _SLICE)])
    pltpu.sync_copy(zbuf, ecnt_s.at[pl.ds(off, _SLICE)])
    pltpu.sync_copy(zbuf, rsum_s.at[pl.ds(off, _SLICE)])
    pltpu.sync_copy(zbuf, rcnt_s.at[pl.ds(off, _SLICE)])

    pltpu.sync_copy(posF.at[pl.ds(base, _EPT)], i_ph)
    pltpu.sync_copy(posF.at[pl.ds(_BATCH + base, _EPT)], i_pr)
    pltpu.sync_copy(posF.at[pl.ds(2 * _BATCH + base, _EPT)], i_pt)
    pltpu.sync_copy(negF.at[pl.ds(base, _EPT)], i_nh)
    pltpu.sync_copy(negF.at[pl.ds(_BATCH + base, _EPT)], i_nr)
    pltpu.sync_copy(negF.at[pl.ds(2 * _BATCH + base, _EPT)], i_nt)

    plsc.subcore_barrier()

    lane = lax.iota(jnp.int32, 16)
    last = lane == 15

    gdn = lax.GatherDimensionNumbers(
        offset_dims=(), collapsed_slice_dims=(0,), start_index_map=(0,))

    def hsum(x):
        for k in (1, 2, 4, 8):
            perm = lax.gather(x, (lane ^ k)[:, None], gdn, slice_sizes=(1,),
                              mode=lax.GatherScatterMode.PROMISE_IN_BOUNDS)
            x = x + perm
        return x

    def edge_body(e, carry):
        dacc = zero16
        hacc = zero16
        tacc = zero16
        racc = zero16
        for j in range(_GROUPS):
            hv = rows_h[e, pl.ds(j * 16, 16)]
            rv = rows_r[e, pl.ds(j * 16, 16)]
            tv = rows_t[e, pl.ds(j * 16, 16)]
            d = hv + rv - tv
            dacc = dacc + d * d
            hacc = hacc + hv * hv
            tacc = tacc + tv * tv
            racc = racc + rv * rv
        eidx = jnp.full((16,), e, jnp.int32)
        plsc.store_scatter(v_d, [eidx], hsum(dacc), mask=last)
        plsc.store_scatter(v_h, [eidx], hsum(hacc), mask=last)
        plsc.store_scatter(v_t, [eidx], hsum(tacc), mask=last)
        plsc.store_scatter(v_r, [eidx], hsum(racc), mask=last)
        return carry

    pltpu.sync_copy(ent.at[i_ph], rows_h)
    pltpu.sync_copy(rel.at[i_pr], rows_r)
    pltpu.sync_copy(ent.at[i_pt], rows_t)
    lax.fori_loop(0, _EPT, edge_body, None)
    pltpu.sync_copy(v_d, posd_o.at[pl.ds(base, _EPT)])
    pltpu.sync_copy(v_h, esum_s.at[i_ph], add=True)
    pltpu.sync_copy(ones_v, ecnt_s.at[i_ph], add=True)
    pltpu.sync_copy(v_t, esum_s.at[i_pt], add=True)
    pltpu.sync_copy(ones_v, ecnt_s.at[i_pt], add=True)
    pltpu.sync_copy(v_r, rsum_s.at[i_pr], add=True)
    pltpu.sync_copy(ones_v, rcnt_s.at[i_pr], add=True)

    pltpu.sync_copy(ent.at[i_nh], rows_h)
    pltpu.sync_copy(rel.at[i_nr], rows_r)
    pltpu.sync_copy(ent.at[i_nt], rows_t)
    lax.fori_loop(0, _EPT, edge_body, None)
    pltpu.sync_copy(v_d, negd_o.at[pl.ds(base, _EPT)])
    pltpu.sync_copy(v_h, esum_s.at[i_nh], add=True)
    pltpu.sync_copy(ones_v, ecnt_s.at[i_nh], add=True)
    pltpu.sync_copy(v_t, esum_s.at[i_nt], add=True)
    pltpu.sync_copy(ones_v, ecnt_s.at[i_nt], add=True)

    plsc.subcore_barrier()

    pltpu.sync_copy(esum_s.at[pl.ds(off, _SLICE)], esum_o.at[c, pl.ds(off, _SLICE)])
    pltpu.sync_copy(ecnt_s.at[pl.ds(off, _SLICE)], ecnt_o.at[c, pl.ds(off, _SLICE)])
    pltpu.sync_copy(rsum_s.at[pl.ds(off, _SLICE)], rsum_o.at[c, pl.ds(off, _SLICE)])
    pltpu.sync_copy(rcnt_s.at[pl.ds(off, _SLICE)], rcnt_o.at[c, pl.ds(off, _SLICE)])


def _tc_reduce(pd, nd, es, ec, rs, rc, out):
    pos = jnp.sqrt(pd[...])
    neg = jnp.sqrt(nd[...])
    main = jnp.sum(jnp.maximum(1.0 + pos - neg, 0.0))

    def scale_loss(sum_ref, cnt_ref):
        tot = sum_ref[0] + sum_ref[1]
        cnt = cnt_ref[0] + cnt_ref[1]
        pres = cnt > 0.5
        val = jnp.sqrt(tot / jnp.maximum(cnt, 1.0)) - 1.0
        num = jnp.sum(jnp.where(pres, jnp.maximum(val, 0.0), 0.0))
        den = jnp.sum(jnp.where(pres, 1.0, 0.0))
        return num / den

    total = main + scale_loss(es, ec) + scale_loss(rs, rc)
    out[...] = jnp.reshape(total, (1, 1))


@jax.jit
def _impl(pos_edge, neg_edge, entity_emb, relation_emb):
    posF = jnp.asarray(pos_edge, jnp.int32).T.reshape(-1)
    negF = jnp.asarray(neg_edge, jnp.int32).T.reshape(-1)

    mesh = plsc.VectorSubcoreMesh(core_axis_name="c", subcore_axis_name="s")
    f32 = jnp.float32
    sc = pl.kernel(
        _sc_body,
        out_type=[
            jax.ShapeDtypeStruct((_BATCH,), f32),
            jax.ShapeDtypeStruct((_BATCH,), f32),
            jax.ShapeDtypeStruct((2, _PAD), f32),
            jax.ShapeDtypeStruct((2, _PAD), f32),
            jax.ShapeDtypeStruct((2, _PAD), f32),
            jax.ShapeDtypeStruct((2, _PAD), f32),
        ],
        mesh=mesh,
        compiler_params=pltpu.CompilerParams(needs_layout_passes=False),
        scratch_types=[
            pltpu.VMEM((_EPT,), jnp.int32),
            pltpu.VMEM((_EPT,), jnp.int32),
            pltpu.VMEM((_EPT,), jnp.int32),
            pltpu.VMEM((_EPT,), jnp.int32),
            pltpu.VMEM((_EPT,), jnp.int32),
            pltpu.VMEM((_EPT,), jnp.int32),
            pltpu.VMEM((_EPT, _EMB_DIM), f32),
            pltpu.VMEM((_EPT, _EMB_DIM), f32),
            pltpu.VMEM((_EPT, _EMB_DIM), f32),
            pltpu.VMEM((_EPT,), f32),
            pltpu.VMEM((_EPT,), f32),
            pltpu.VMEM((_EPT,), f32),
            pltpu.VMEM((_EPT,), f32),
            pltpu.VMEM((_EPT,), f32),
            pltpu.VMEM((_SLICE,), f32),
            pltpu.VMEM_SHARED((_PAD,), f32),
            pltpu.VMEM_SHARED((_PAD,), f32),
            pltpu.VMEM_SHARED((_PAD,), f32),
            pltpu.VMEM_SHARED((_PAD,), f32),
        ],
    )
    pd, nd, es, ec, rs, rc = sc(posF, negF, entity_emb, relation_emb)

    red = pl.pallas_call(
        _tc_reduce,
        out_shape=jax.ShapeDtypeStruct((1, 1), f32),
    )
    loss = red(
        pd.reshape(32, 128), nd.reshape(32, 128),
        es.reshape(2, _PAD // 128, 128), ec.reshape(2, _PAD // 128, 128),
        rs.reshape(2, _PAD // 128, 128), rc.reshape(2, _PAD // 128, 128),
    )
    return jnp.reshape(loss, ())


def kernel(pos_edge, neg_edge, entity_emb, relation_emb):
    return _impl(pos_edge, neg_edge, entity_emb, relation_emb)


# speedup vs baseline: 2.0505x; 1.1720x over previous
import jax
import jax.numpy as jnp
from jax import lax
from jax.experimental import pallas as pl
from jax.experimental.pallas import tpu as pltpu
from jax.experimental.pallas import tpu_sc as plsc

_EMB_DIM = 128
_BATCH = 4096
_PAD = 100352
_TILES = 32
_EPT = _BATCH // _TILES
_SLICE = _PAD // 16
_GROUPS = _EMB_DIM // 16


def _sc_body(posI, negI, ent, rel,
             main_o, esum_o, ecnt_o, rsum_o, rcnt_o,
             idx_p, idx_n,
             hp, rp, tp, hn, rn, tn,
             vh_p, vt_p, vr_p, vh_n, vt_n,
             ones_v, mbuf, zbuf,
             esum_s, ecnt_s, rsum_s, rcnt_s,
             sem_g, sem_i, sem_s):
    c = lax.axis_index("c")
    s = lax.axis_index("s")
    wid = c * 16 + s

    zero16 = jnp.zeros((16,), jnp.float32)
    one16 = jnp.ones((16,), jnp.float32)
    lane = lax.iota(jnp.int32, 16)
    last = lane == 15

    pltpu.sync_copy(posI.at[wid], idx_p)
    pltpu.sync_copy(negI.at[wid], idx_n)
    g0 = pltpu.async_copy(ent.at[idx_p.at[0]], hp, sem_g)
    g1 = pltpu.async_copy(rel.at[idx_p.at[1]], rp, sem_g)
    g2 = pltpu.async_copy(ent.at[idx_p.at[2]], tp, sem_g)
    g3 = pltpu.async_copy(ent.at[idx_n.at[0]], hn, sem_g)
    g4 = pltpu.async_copy(rel.at[idx_n.at[1]], rn, sem_g)
    g5 = pltpu.async_copy(ent.at[idx_n.at[2]], tn, sem_g)

    def zfill(i, carry):
        zbuf[pl.ds(i * 16, 16)] = zero16
        return carry

    lax.fori_loop(0, _SLICE // 64, zfill, None)
    off = s * _SLICE
    inits = []
    for arr in (esum_s, ecnt_s, rsum_s, rcnt_s):
        for q in range(4):
            inits.append(pltpu.async_copy(
                zbuf, arr.at[pl.ds(off + q * (_SLICE // 4), _SLICE // 4)],
                sem_i))
    for i in range(_EPT // 16):
        ones_v[pl.ds(i * 16, 16)] = one16

    gdn = lax.GatherDimensionNumbers(
        offset_dims=(), collapsed_slice_dims=(0,), start_index_map=(0,))

    def hsum(x):
        for k in (1, 2, 4, 8):
            perm = lax.gather(x, (lane ^ k)[:, None], gdn, slice_sizes=(1,),
                              mode=lax.GatherScatterMode.PROMISE_IN_BOUNDS)
            x = x + perm
        return x

    def vsqrt(x):
        i = plsc.bitcast(x, jnp.int32)
        y = plsc.bitcast(jnp.int32(0x1FBD1DF5) + (i >> 1), jnp.float32)
        for _ in range(3):
            y = 0.5 * (y + x / y)
        return y

    for g in (g0, g1, g2, g3, g4, g5):
        g.wait()
    for i in inits:
        i.wait()

    def edge_body(e, macc):
        dps = dns = hps = tps = rps = hns = tns = zero16
        for j in range(_GROUPS):
            col = pl.ds(j * 16, 16)
            hpv = hp[e, col]
            rpv = rp[e, col]
            tpv = tp[e, col]
            hnv = hn[e, col]
            rnv = rn[e, col]
            tnv = tn[e, col]
            dp = hpv + rpv - tpv
            dn = hnv + rnv - tnv
            dps = dps + dp * dp
            dns = dns + dn * dn
            hps = hps + hpv * hpv
            tps = tps + tpv * tpv
            rps = rps + rpv * rpv
            hns = hns + hnv * hnv
            tns = tns + tnv * tnv
        eidx = jnp.full((16,), e, jnp.int32)
        plsc.store_scatter(vh_p, [eidx], hsum(hps), mask=last)
        plsc.store_scatter(vt_p, [eidx], hsum(tps), mask=last)
        plsc.store_scatter(vr_p, [eidx], hsum(rps), mask=last)
        plsc.store_scatter(vh_n, [eidx], hsum(hns), mask=last)
        plsc.store_scatter(vt_n, [eidx], hsum(tns), mask=last)
        contrib = jnp.maximum(1.0 + vsqrt(hsum(dps)) - vsqrt(hsum(dns)), 0.0)
        return macc + jnp.where(last, contrib, 0.0)

    macc = lax.fori_loop(0, _EPT, edge_body, zero16)
    mbuf[pl.ds(0, 16)] = macc
    m0 = pltpu.async_copy(mbuf, main_o.at[wid], sem_i)

    plsc.subcore_barrier()

    s0 = pltpu.async_copy(vh_p, esum_s.at[idx_p.at[0]], sem_s, add=True)
    s1 = pltpu.async_copy(ones_v, ecnt_s.at[idx_p.at[0]], sem_s, add=True)
    s2 = pltpu.async_copy(vt_p, esum_s.at[idx_p.at[2]], sem_s, add=True)
    s3 = pltpu.async_copy(ones_v, ecnt_s.at[idx_p.at[2]], sem_s, add=True)
    s4 = pltpu.async_copy(vh_n, esum_s.at[idx_n.at[0]], sem_s, add=True)
    s5 = pltpu.async_copy(ones_v, ecnt_s.at[idx_n.at[0]], sem_s, add=True)
    s6 = pltpu.async_copy(vt_n, esum_s.at[idx_n.at[2]], sem_s, add=True)
    s7 = pltpu.async_copy(ones_v, ecnt_s.at[idx_n.at[2]], sem_s, add=True)
    s8 = pltpu.async_copy(vr_p, rsum_s.at[idx_p.at[1]], sem_s, add=True)
    s9 = pltpu.async_copy(ones_v, rcnt_s.at[idx_p.at[1]], sem_s, add=True)
    for sd in (s0, s1, s2, s3, s4, s5, s6, s7, s8, s9):
        sd.wait()
    m0.wait()

    plsc.subcore_barrier()

    pltpu.sync_copy(esum_s.at[pl.ds(off, _SLICE)], esum_o.at[c, pl.ds(off, _SLICE)])
    pltpu.sync_copy(ecnt_s.at[pl.ds(off, _SLICE)], ecnt_o.at[c, pl.ds(off, _SLICE)])
    pltpu.sync_copy(rsum_s.at[pl.ds(off, _SLICE)], rsum_o.at[c, pl.ds(off, _SLICE)])
    pltpu.sync_copy(rcnt_s.at[pl.ds(off, _SLICE)], rcnt_o.at[c, pl.ds(off, _SLICE)])


def _tc_reduce(mo, es, ec, rs, rc, out):
    main = jnp.sum(mo[...])

    def scale_loss(sum_ref, cnt_ref):
        tot = sum_ref[0] + sum_ref[1]
        cnt = cnt_ref[0] + cnt_ref[1]
        pres = cnt > 0.5
        val = jnp.sqrt(tot / jnp.maximum(cnt, 1.0)) - 1.0
        num = jnp.sum(jnp.where(pres, jnp.maximum(val, 0.0), 0.0))
        den = jnp.sum(jnp.where(pres, 1.0, 0.0))
        return num / den

    total = main + scale_loss(es, ec) + scale_loss(rs, rc)
    out[...] = jnp.reshape(total, (1, 1))


@jax.jit
def _impl(pos_edge, neg_edge, entity_emb, relation_emb):
    posI = jnp.asarray(pos_edge, jnp.int32).T.reshape(3, _TILES, _EPT)
    posI = posI.transpose(1, 0, 2)
    negI = jnp.asarray(neg_edge, jnp.int32).T.reshape(3, _TILES, _EPT)
    negI = negI.transpose(1, 0, 2)

    mesh = plsc.VectorSubcoreMesh(core_axis_name="c", subcore_axis_name="s")
    f32 = jnp.float32
    sc = pl.kernel(
        _sc_body,
        out_type=[
            jax.ShapeDtypeStruct((_TILES, 16), f32),
            jax.ShapeDtypeStruct((2, _PAD), f32),
            jax.ShapeDtypeStruct((2, _PAD), f32),
            jax.ShapeDtypeStruct((2, _PAD), f32),
            jax.ShapeDtypeStruct((2, _PAD), f32),
        ],
        mesh=mesh,
        compiler_params=pltpu.CompilerParams(needs_layout_passes=False),
        scratch_types=[
            pltpu.VMEM((3, _EPT), jnp.int32),
            pltpu.VMEM((3, _EPT), jnp.int32),
            pltpu.VMEM((_EPT, _EMB_DIM), f32),
            pltpu.VMEM((_EPT, _EMB_DIM), f32),
            pltpu.VMEM((_EPT, _EMB_DIM), f32),
            pltpu.VMEM((_EPT, _EMB_DIM), f32),
            pltpu.VMEM((_EPT, _EMB_DIM), f32),
            pltpu.VMEM((_EPT, _EMB_DIM), f32),
            pltpu.VMEM((_EPT,), f32),
            pltpu.VMEM((_EPT,), f32),
            pltpu.VMEM((_EPT,), f32),
            pltpu.VMEM((_EPT,), f32),
            pltpu.VMEM((_EPT,), f32),
            pltpu.VMEM((_EPT,), f32),
            pltpu.VMEM((16,), f32),
            pltpu.VMEM((_SLICE // 4,), f32),
            pltpu.VMEM_SHARED((_PAD,), f32),
            pltpu.VMEM_SHARED((_PAD,), f32),
            pltpu.VMEM_SHARED((_PAD,), f32),
            pltpu.VMEM_SHARED((_PAD,), f32),
            pltpu.SemaphoreType.DMA,
            pltpu.SemaphoreType.DMA,
            pltpu.SemaphoreType.DMA,
        ],
    )
    mo, es, ec, rs, rc = sc(posI, negI, entity_emb, relation_emb)

    red = pl.pallas_call(
        _tc_reduce,
        out_shape=jax.ShapeDtypeStruct((1, 1), f32),
    )
    loss = red(
        mo,
        es.reshape(2, _PAD // 128, 128), ec.reshape(2, _PAD // 128, 128),
        rs.reshape(2, _PAD // 128, 128), rc.reshape(2, _PAD // 128, 128),
    )
    return jnp.reshape(loss, ())


def kernel(pos_edge, neg_edge, entity_emb, relation_emb):
    return _impl(pos_edge, neg_edge, entity_emb, relation_emb)
